# baseline (speedup 1.0000x reference)
import functools

import numpy as np
import jax
import jax.numpy as jnp
from jax import lax
from jax.experimental import pallas as pl
from jax.experimental.pallas import tpu as pltpu
from jax.experimental.pallas import tpu_sc as plsc

H = 8
C = 16
HC = H * C
N_NODES = 10000
N_EDGES = 320000
D_IN = 128

NC = 2
NS = 16
NW = NC * NS
L = 16

EDGES_PER_W = N_EDGES // NW
BLK = 40
NBLK = EDGES_PER_W // BLK
SB = 10
NPAD = 10240
ZROWS = NPAD // NS
NDEN = NPAD // 8
ZDROWS = NDEN // NS


def _proj_body(x_ref, kw, kb, qw, qb, vw, vb, k_out, qv_out):
    xb = x_ref[...]
    k_out[...] = jnp.dot(xb, kw[...], preferred_element_type=jnp.float32) + kb[...]
    qv_out[:, :HC] = jnp.dot(xb, qw[...], preferred_element_type=jnp.float32) + qb[...]
    qv_out[:, HC:] = jnp.dot(xb, vw[...], preferred_element_type=jnp.float32) + vb[...]


def _node_proj(x, Kw, Kb, Qw, Qb, Vw, Vb):
    n = x.shape[0]
    bn = 2000
    grid = n // bn
    full = pl.BlockSpec((D_IN, HC), lambda i: (0, 0))
    bias = pl.BlockSpec((1, HC), lambda i: (0, 0))
    return pl.pallas_call(
        _proj_body,
        grid=(grid,),
        in_specs=[pl.BlockSpec((bn, D_IN), lambda i: (i, 0)),
                  full, bias, full, bias, full, bias],
        out_specs=[pl.BlockSpec((bn, HC), lambda i: (i, 0)),
                   pl.BlockSpec((bn, 2 * HC), lambda i: (i, 0))],
        out_shape=[jax.ShapeDtypeStruct((n, HC), jnp.float32),
                   jax.ShapeDtypeStruct((n, 2 * HC), jnp.float32)],
    )(x, Kw, Kb.reshape(1, HC), Qw, Qb.reshape(1, HC), Vw, Vb.reshape(1, HC))


def _eproj_body(e_ref, w, b, out):
    out[...] = jnp.dot(e_ref[...], w[...], preferred_element_type=jnp.float32) + b[...]


def _edge_proj(e, Ew, Eb):
    m = e.shape[0]
    bm = 4000
    grid = m // bm
    return pl.pallas_call(
        _eproj_body,
        grid=(grid,),
        in_specs=[pl.BlockSpec((bm, D_IN), lambda i: (i, 0)),
                  pl.BlockSpec((D_IN, HC), lambda i: (0, 0)),
                  pl.BlockSpec((1, HC), lambda i: (0, 0))],
        out_specs=pl.BlockSpec((bm, HC), lambda i: (i, 0)),
        out_shape=jax.ShapeDtypeStruct((m, HC), jnp.float32),
    )(e, Ew, Eb.reshape(1, HC))


_SC_MESH = plsc.VectorSubcoreMesh(
    core_axis_name="c", subcore_axis_name="s", num_cores=NC, num_subcores=NS)


@functools.partial(
    pl.kernel,
    out_type=(
        jax.ShapeDtypeStruct((N_EDGES, HC), jnp.float32),
        jax.ShapeDtypeStruct((NC, NPAD, HC), jnp.float32),
        jax.ShapeDtypeStruct((NC, NDEN, HC), jnp.float32),
    ),
    mesh=_SC_MESH,
    scratch_types=[
        pltpu.VMEM((SB * BLK,), jnp.int32),
        pltpu.VMEM((SB * BLK,), jnp.int32),
        pltpu.VMEM((BLK,), jnp.int32),
        pltpu.VMEM((BLK,), jnp.int32),
        pltpu.VMEM((BLK, HC), jnp.float32),
        pltpu.VMEM((BLK, 2 * HC), jnp.float32),
        pltpu.VMEM((BLK, HC), jnp.float32),
        pltpu.VMEM((BLK, HC), jnp.float32),
        pltpu.VMEM((BLK, HC), jnp.float32),
        pltpu.VMEM((BLK, HC), jnp.float32),
        pltpu.VMEM_SHARED((NPAD, HC), jnp.float32),
        pltpu.VMEM_SHARED((NDEN, HC), jnp.float32),
        pltpu.SemaphoreType.DMA,
        pltpu.SemaphoreType.DMA,
        pltpu.SemaphoreType.DMA,
    ],
)
def _edge_kernel(kh_hbm, qv_hbm, ee_hbm, src_hbm, dst_hbm, z_hbm,
                 eout_hbm, partn_hbm, partd_hbm,
                 src_sb, dst_sb, src_v, src8_v, k_v, qv_v, ee_v, eout_v,
                 rown_v, rowd_v, accn, accd, sem0, sem1, sem2):
    cid = lax.axis_index("c")
    sid = lax.axis_index("s")
    wid = cid * NS + sid

    pltpu.sync_copy(z_hbm, accn.at[pl.ds(sid * ZROWS, ZROWS)])
    pltpu.sync_copy(z_hbm.at[pl.ds(0, ZDROWS)],
                    accd.at[pl.ds(sid * ZDROWS, ZDROWS)])
    plsc.subcore_barrier()

    lane = lax.broadcasted_iota(jnp.int32, (L,), 0)
    perms = [jnp.bitwise_xor(lane, m) for m in (1, 2, 4, 8)]
    zero16 = jnp.zeros((L,), jnp.float32)
    one_i = jnp.ones((L,), jnp.int32)
    lane_onehot = [
        (one_i - jnp.minimum(jnp.bitwise_xor(lane, h), one_i)
         ).astype(jnp.float32)
        for h in range(H)]

    def lane_sum_splat(v):
        for p in perms:
            v = v + v.at[p].get(mode="promise_in_bounds", unique_indices=True)
        return v

    def edge_body(i, carry):
        dvec = zero16
        for h in range(H):
            sl = pl.ds(h * C, C)
            eo = k_v[i, sl] * qv_v[i, sl] * ee_v[i, sl] * 0.25
            eout_v[i, sl] = eo
            exv = jnp.exp(lane_sum_splat(eo))
            rown_v[i, sl] = exv * qv_v[i, pl.ds(HC + h * C, C)]
            dvec = dvec + exv * lane_onehot[h]
        ib = pl.multiple_of(jnp.minimum((i // L) * L, BLK - L), 8)
        off = i - ib
        chunk = src_v[pl.ds(ib, L)]
        slotv = chunk.at[jnp.broadcast_to(off, (L,))].get(
            mode="promise_in_bounds") & 7
        for j in range(8):
            mf = (one_i - jnp.minimum(jnp.bitwise_xor(slotv, j), one_i)
                  ).astype(jnp.float32)
            rowd_v[i, pl.ds(j * L, L)] = dvec * mf
        return carry

    _windows = sorted({min(t * L, BLK - L) for t in range((BLK + L - 1) // L)})

    def conv_idx(b):
        sboff = pl.multiple_of((b % SB) * BLK, 8)
        for w in _windows:
            s = src_sb[pl.ds(sboff + w, L)]
            src_v[pl.ds(w, L)] = s
            src8_v[pl.ds(w, L)] = lax.shift_right_logical(s, 3)

    def blk_body(b, carry):
        base = wid * EDGES_PER_W + b * BLK

        @pl.when(b % SB == 0)
        def _():
            pltpu.sync_copy(src_hbm.at[pl.ds(base, SB * BLK)], src_sb)
            pltpu.sync_copy(dst_hbm.at[pl.ds(base, SB * BLK)], dst_sb)

        sboff = pl.multiple_of((b % SB) * BLK, 8)
        cpk = pltpu.async_copy(kh_hbm.at[src_sb.at[pl.ds(sboff, BLK)]],
                               k_v, sem0)
        cpq = pltpu.async_copy(qv_hbm.at[dst_sb.at[pl.ds(sboff, BLK)]],
                               qv_v, sem1)
        pltpu.sync_copy(ee_hbm.at[pl.ds(base, BLK)], ee_v)
        conv_idx(b)
        cpk.wait()
        cpq.wait()
        lax.fori_loop(0, BLK, edge_body, 0)
        pltpu.sync_copy(eout_v, eout_hbm.at[pl.ds(base, BLK)])
        pltpu.sync_copy(rown_v, accn.at[src_v], add=True)
        pltpu.sync_copy(rowd_v, accd.at[src8_v], add=True)
        return carry

    lax.fori_loop(0, NBLK, blk_body, 0)

    plsc.subcore_barrier()
    pltpu.sync_copy(accn.at[pl.ds(sid * ZROWS, ZROWS)],
                    partn_hbm.at[cid, pl.ds(sid * ZROWS, ZROWS)])
    pltpu.sync_copy(accd.at[pl.ds(sid * ZDROWS, ZDROWS)],
                    partd_hbm.at[cid, pl.ds(sid * ZDROWS, ZDROWS)])


def _combine_body(n0, n1, d0, d1, m, out):
    num = n0[...] + n1[...]
    den = jnp.dot(d0[...] + d1[...], m[...], preferred_element_type=jnp.float32)
    out[...] = num / jnp.maximum(den, 1e-30)


def _combine(p0n, p1n, p0d, p1d, onehot):
    n = p0n.shape[0]
    bn = 2000
    grid = n // bn
    rows = pl.BlockSpec((bn, HC), lambda i: (i, 0))
    drows = pl.BlockSpec((bn, H), lambda i: (i, 0))
    return pl.pallas_call(
        _combine_body,
        grid=(grid,),
        in_specs=[rows, rows, drows, drows,
                  pl.BlockSpec((H, HC), lambda i: (0, 0))],
        out_specs=rows,
        out_shape=jax.ShapeDtypeStruct((n, HC), jnp.float32),
    )(p0n, p1n, p0d, p1d, onehot)


def kernel(x, e, edge_index, Qw, Qb, Kw, Kb, Vw, Vb, Ew, Eb):
    src = edge_index[0].astype(jnp.int32)
    dst = edge_index[1].astype(jnp.int32)

    Kh, QVh = _node_proj(x, Kw, Kb, Qw, Qb, Vw, Vb)
    Ee = _edge_proj(e, Ew, Eb)

    zrows = jnp.zeros((ZROWS, HC), jnp.float32)
    e_out, partn, partd = _edge_kernel(Kh, QVh, Ee, src, dst, zrows)

    p0n = partn[0, :N_NODES, :]
    p1n = partn[1, :N_NODES, :]
    p0d = partd[0].reshape(NDEN * 8, L)[:N_NODES, :H]
    p1d = partd[1].reshape(NDEN * 8, L)[:N_NODES, :H]
    onehot = jnp.repeat(jnp.eye(H, dtype=jnp.float32), C, axis=1)
    h_out = _combine(p0n, p1n, p0d, p1d, onehot)
    return (h_out, e_out)


# speedup vs baseline: 2.5420x; 2.5420x over previous
import functools

import numpy as np
import jax
import jax.numpy as jnp
from jax import lax
from jax.experimental import pallas as pl
from jax.experimental.pallas import tpu as pltpu
from jax.experimental.pallas import tpu_sc as plsc

H = 8
C = 16
HC = H * C
N_NODES = 10000
N_EDGES = 320000
D_IN = 128

NC = 2
NS = 16
NW = NC * NS
L = 16

EDGES_PER_W = N_EDGES // NW
BLK = 40
NBLK = EDGES_PER_W // BLK
SB = 10
NPAD = 10240
ZROWS = NPAD // NS
NDEN = NPAD // 8
ZDROWS = NDEN // NS


def _proj_body(x_ref, kw, kb, qw, qb, vw, vb, k_out, q_out, v_out):
    xb = x_ref[...]
    k_out[...] = jnp.dot(xb, kw[...], preferred_element_type=jnp.float32) + kb[...]
    q_out[...] = jnp.dot(xb, qw[...], preferred_element_type=jnp.float32) + qb[...]
    v_out[...] = jnp.dot(xb, vw[...], preferred_element_type=jnp.float32) + vb[...]


def _node_proj(x, Kw, Kb, Qw, Qb, Vw, Vb):
    n = x.shape[0]
    bn = 2000
    grid = n // bn
    full = pl.BlockSpec((D_IN, HC), lambda i: (0, 0))
    bias = pl.BlockSpec((1, HC), lambda i: (0, 0))
    rows = pl.BlockSpec((bn, HC), lambda i: (i, 0))
    return pl.pallas_call(
        _proj_body,
        grid=(grid,),
        in_specs=[pl.BlockSpec((bn, D_IN), lambda i: (i, 0)),
                  full, bias, full, bias, full, bias],
        out_specs=[rows, rows, rows],
        out_shape=[jax.ShapeDtypeStruct((n, HC), jnp.float32)] * 3,
    )(x, Kw, Kb.reshape(1, HC), Qw, Qb.reshape(1, HC), Vw, Vb.reshape(1, HC))


def _eproj_body(e_ref, w, b, out):
    out[...] = jnp.dot(e_ref[...], w[...], preferred_element_type=jnp.float32) + b[...]


def _edge_proj(e, Ew, Eb):
    m = e.shape[0]
    bm = 4000
    grid = m // bm
    return pl.pallas_call(
        _eproj_body,
        grid=(grid,),
        in_specs=[pl.BlockSpec((bm, D_IN), lambda i: (i, 0)),
                  pl.BlockSpec((D_IN, HC), lambda i: (0, 0)),
                  pl.BlockSpec((1, HC), lambda i: (0, 0))],
        out_specs=pl.BlockSpec((bm, HC), lambda i: (i, 0)),
        out_shape=jax.ShapeDtypeStruct((m, HC), jnp.float32),
    )(e, Ew, Eb.reshape(1, HC))


_SC_MESH = plsc.VectorSubcoreMesh(
    core_axis_name="c", subcore_axis_name="s", num_cores=NC, num_subcores=NS)


@functools.partial(
    pl.kernel,
    out_type=(
        jax.ShapeDtypeStruct((N_EDGES, HC), jnp.float32),
        jax.ShapeDtypeStruct((NC, NPAD, HC), jnp.float32),
        jax.ShapeDtypeStruct((NC, NDEN, HC), jnp.float32),
    ),
    mesh=_SC_MESH,
    scratch_types=[
        pltpu.VMEM((SB * BLK,), jnp.int32),
        pltpu.VMEM((SB * BLK,), jnp.int32),
        pltpu.VMEM((BLK,), jnp.int32),
        pltpu.VMEM((BLK,), jnp.int32),
        pltpu.VMEM((BLK, HC), jnp.float32),
        pltpu.VMEM((BLK, HC), jnp.float32),
        pltpu.VMEM((BLK, HC), jnp.float32),
        pltpu.VMEM((BLK, HC), jnp.float32),
        pltpu.VMEM((BLK, HC), jnp.float32),
        pltpu.VMEM((BLK, HC), jnp.float32),
        pltpu.VMEM((BLK, HC), jnp.float32),
        pltpu.VMEM_SHARED((NPAD, HC), jnp.float32),
        pltpu.VMEM_SHARED((NDEN, HC), jnp.float32),
        pltpu.SemaphoreType.DMA,
        pltpu.SemaphoreType.DMA,
        pltpu.SemaphoreType.DMA,
    ],
)
def _edge_kernel(kh_hbm, qh_hbm, vh_hbm, ee_hbm, src_hbm, dst_hbm, z_hbm,
                 eout_hbm, partn_hbm, partd_hbm,
                 src_sb, dst_sb, src_v, src8_v, k_v, q_v, v_v, ee_v, eout_v,
                 rown_v, rowd_v, accn, accd, sem0, sem1, sem2):
    cid = lax.axis_index("c")
    sid = lax.axis_index("s")
    wid = cid * NS + sid

    pltpu.sync_copy(z_hbm, accn.at[pl.ds(sid * ZROWS, ZROWS)])
    pltpu.sync_copy(z_hbm.at[pl.ds(0, ZDROWS)],
                    accd.at[pl.ds(sid * ZDROWS, ZDROWS)])
    plsc.subcore_barrier()

    lane = lax.broadcasted_iota(jnp.int32, (L,), 0)
    perms = [jnp.bitwise_xor(lane, m) for m in (1, 2, 4, 8)]
    zero16 = jnp.zeros((L,), jnp.float32)
    one_i = jnp.ones((L,), jnp.int32)
    lane_onehot = [
        (one_i - jnp.minimum(jnp.bitwise_xor(lane, h), one_i)
         ).astype(jnp.float32)
        for h in range(H)]

    def lane_sum_splat(v):
        for p in perms:
            v = v + v.at[p].get(mode="promise_in_bounds", unique_indices=True)
        return v

    def edge_body(i, carry):
        dvec = zero16
        for h in range(H):
            sl = pl.ds(h * C, C)
            eo = k_v[i, sl] * q_v[i, sl] * ee_v[i, sl] * 0.25
            eout_v[i, sl] = eo
            exv = jnp.exp(lane_sum_splat(eo))
            rown_v[i, sl] = exv * v_v[i, sl]
            dvec = dvec + exv * lane_onehot[h]
        ib = pl.multiple_of(jnp.minimum((i // L) * L, BLK - L), 8)
        off = i - ib
        chunk = src_v[pl.ds(ib, L)]
        slotv = chunk.at[jnp.broadcast_to(off, (L,))].get(
            mode="promise_in_bounds") & 7
        for j in range(8):
            mf = (one_i - jnp.minimum(jnp.bitwise_xor(slotv, j), one_i)
                  ).astype(jnp.float32)
            rowd_v[i, pl.ds(j * L, L)] = dvec * mf
        return carry

    _windows = sorted({min(t * L, BLK - L) for t in range((BLK + L - 1) // L)})

    def conv_idx(b):
        sboff = pl.multiple_of((b % SB) * BLK, 8)
        for w in _windows:
            s = src_sb[pl.ds(sboff + w, L)]
            src_v[pl.ds(w, L)] = s
            src8_v[pl.ds(w, L)] = lax.shift_right_logical(s, 3)

    def blk_body(b, carry):
        base = wid * EDGES_PER_W + b * BLK

        @pl.when(b % SB == 0)
        def _():
            pltpu.sync_copy(src_hbm.at[pl.ds(base, SB * BLK)], src_sb)
            pltpu.sync_copy(dst_hbm.at[pl.ds(base, SB * BLK)], dst_sb)

        sboff = pl.multiple_of((b % SB) * BLK, 8)
        cpk = pltpu.async_copy(kh_hbm.at[src_sb.at[pl.ds(sboff, BLK)]],
                               k_v, sem0)
        cpq = pltpu.async_copy(qh_hbm.at[dst_sb.at[pl.ds(sboff, BLK)]],
                               q_v, sem1)
        cpv = pltpu.async_copy(vh_hbm.at[dst_sb.at[pl.ds(sboff, BLK)]],
                               v_v, sem2)
        pltpu.sync_copy(ee_hbm.at[pl.ds(base, BLK)], ee_v)
        conv_idx(b)
        cpk.wait()
        cpq.wait()
        cpv.wait()
        lax.fori_loop(0, BLK, edge_body, 0)
        pltpu.sync_copy(eout_v, eout_hbm.at[pl.ds(base, BLK)])
        pltpu.sync_copy(rown_v, accn.at[src_v], add=True)
        pltpu.sync_copy(rowd_v, accd.at[src8_v], add=True)
        return carry

    lax.fori_loop(0, NBLK, blk_body, 0)

    plsc.subcore_barrier()
    pltpu.sync_copy(accn.at[pl.ds(sid * ZROWS, ZROWS)],
                    partn_hbm.at[cid, pl.ds(sid * ZROWS, ZROWS)])
    pltpu.sync_copy(accd.at[pl.ds(sid * ZDROWS, ZDROWS)],
                    partd_hbm.at[cid, pl.ds(sid * ZDROWS, ZDROWS)])


def _combine_body(n0, n1, d0, d1, m, out):
    num = n0[...] + n1[...]
    den = jnp.dot(d0[...] + d1[...], m[...], preferred_element_type=jnp.float32)
    out[...] = num / jnp.maximum(den, 1e-30)


def _combine(p0n, p1n, p0d, p1d, onehot):
    n = p0n.shape[0]
    bn = 2000
    grid = n // bn
    rows = pl.BlockSpec((bn, HC), lambda i: (i, 0))
    drows = pl.BlockSpec((bn, H), lambda i: (i, 0))
    return pl.pallas_call(
        _combine_body,
        grid=(grid,),
        in_specs=[rows, rows, drows, drows,
                  pl.BlockSpec((H, HC), lambda i: (0, 0))],
        out_specs=rows,
        out_shape=jax.ShapeDtypeStruct((n, HC), jnp.float32),
    )(p0n, p1n, p0d, p1d, onehot)


def kernel(x, e, edge_index, Qw, Qb, Kw, Kb, Vw, Vb, Ew, Eb):
    src = edge_index[0].astype(jnp.int32)
    dst = edge_index[1].astype(jnp.int32)

    Kh, Qh, Vh = _node_proj(x, Kw, Kb, Qw, Qb, Vw, Vb)
    Ee = _edge_proj(e, Ew, Eb)

    zrows = jnp.zeros((ZROWS, HC), jnp.float32)
    e_out, partn, partd = _edge_kernel(Kh, Qh, Vh, Ee, src, dst, zrows)

    p0n = partn[0, :N_NODES, :]
    p1n = partn[1, :N_NODES, :]
    p0d = partd[0].reshape(NDEN * 8, L)[:N_NODES, :H]
    p1d = partd[1].reshape(NDEN * 8, L)[:N_NODES, :H]
    onehot = jnp.repeat(jnp.eye(H, dtype=jnp.float32), C, axis=1)
    h_out = _combine(p0n, p1n, p0d, p1d, onehot)
    return (h_out, e_out)


# speedup vs baseline: 2.6610x; 1.0468x over previous
import functools

import numpy as np
import jax
import jax.numpy as jnp
from jax import lax
from jax.experimental import pallas as pl
from jax.experimental.pallas import tpu as pltpu
from jax.experimental.pallas import tpu_sc as plsc

H = 8
C = 16
HC = H * C
N_NODES = 10000
N_EDGES = 320000
D_IN = 128

NC = 2
NS = 16
NW = NC * NS
L = 16

EDGES_PER_W = N_EDGES // NW
BLK = 40
NBLK = EDGES_PER_W // BLK
SB = 10
NPAD = 10240
ZROWS = NPAD // NS
NDEN = NPAD // 8
ZDROWS = NDEN // NS


def _proj_body(x_ref, kw, kb, qw, qb, vw, vb, k_out, q_out, v_out):
    xb = x_ref[...]
    k_out[...] = jnp.dot(xb, kw[...], preferred_element_type=jnp.float32) + kb[...]
    q_out[...] = jnp.dot(xb, qw[...], preferred_element_type=jnp.float32) + qb[...]
    v_out[...] = jnp.dot(xb, vw[...], preferred_element_type=jnp.float32) + vb[...]


def _node_proj(x, Kw, Kb, Qw, Qb, Vw, Vb):
    n = x.shape[0]
    bn = 2000
    grid = n // bn
    full = pl.BlockSpec((D_IN, HC), lambda i: (0, 0))
    bias = pl.BlockSpec((1, HC), lambda i: (0, 0))
    rows = pl.BlockSpec((bn, HC), lambda i: (i, 0))
    return pl.pallas_call(
        _proj_body,
        grid=(grid,),
        in_specs=[pl.BlockSpec((bn, D_IN), lambda i: (i, 0)),
                  full, bias, full, bias, full, bias],
        out_specs=[rows, rows, rows],
        out_shape=[jax.ShapeDtypeStruct((n, HC), jnp.float32)] * 3,
    )(x, Kw, Kb.reshape(1, HC), Qw, Qb.reshape(1, HC), Vw, Vb.reshape(1, HC))


def _eproj_body(e_ref, w, b, out):
    out[...] = jnp.dot(e_ref[...], w[...], preferred_element_type=jnp.float32) + b[...]


def _edge_proj(e, Ew, Eb):
    m = e.shape[0]
    bm = 4000
    grid = m // bm
    return pl.pallas_call(
        _eproj_body,
        grid=(grid,),
        in_specs=[pl.BlockSpec((bm, D_IN), lambda i: (i, 0)),
                  pl.BlockSpec((D_IN, HC), lambda i: (0, 0)),
                  pl.BlockSpec((1, HC), lambda i: (0, 0))],
        out_specs=pl.BlockSpec((bm, HC), lambda i: (i, 0)),
        out_shape=jax.ShapeDtypeStruct((m, HC), jnp.float32),
    )(e, Ew, Eb.reshape(1, HC))


_SC_MESH = plsc.VectorSubcoreMesh(
    core_axis_name="c", subcore_axis_name="s", num_cores=NC, num_subcores=NS)


@functools.partial(
    pl.kernel,
    out_type=(
        jax.ShapeDtypeStruct((N_EDGES, HC), jnp.float32),
        jax.ShapeDtypeStruct((NC, NPAD, HC), jnp.float32),
        jax.ShapeDtypeStruct((NC, NDEN, HC), jnp.float32),
    ),
    mesh=_SC_MESH,
    scratch_types=[
        pltpu.VMEM((SB * BLK,), jnp.int32),
        pltpu.VMEM((SB * BLK,), jnp.int32),
        pltpu.VMEM((BLK,), jnp.int32),
        pltpu.VMEM((BLK,), jnp.int32),
        pltpu.VMEM((BLK, HC), jnp.float32),
        pltpu.VMEM((BLK, HC), jnp.float32),
        pltpu.VMEM((BLK, HC), jnp.float32),
        pltpu.VMEM((BLK, HC), jnp.float32),
        pltpu.VMEM((BLK, HC), jnp.float32),
        pltpu.VMEM((BLK, HC), jnp.float32),
        pltpu.VMEM((BLK, HC), jnp.float32),
        pltpu.VMEM_SHARED((NPAD, HC), jnp.float32),
        pltpu.VMEM_SHARED((NDEN, HC), jnp.float32),
        pltpu.SemaphoreType.DMA,
        pltpu.SemaphoreType.DMA,
        pltpu.SemaphoreType.DMA,
        pltpu.SemaphoreType.DMA,
    ],
)
def _edge_kernel(kh_hbm, qh_hbm, vh_hbm, ee_hbm, src_hbm, dst_hbm, z_hbm,
                 eout_hbm, partn_hbm, partd_hbm,
                 src_sb, dst_sb, src_v, src8_v, k_v, q_v, v_v, ee_v, eout_v,
                 rown_v, rowd_v, accn, accd, sem0, sem1, sem2, semo):
    cid = lax.axis_index("c")
    sid = lax.axis_index("s")
    wid = cid * NS + sid

    pltpu.sync_copy(z_hbm, accn.at[pl.ds(sid * ZROWS, ZROWS)])
    pltpu.sync_copy(z_hbm.at[pl.ds(0, ZDROWS)],
                    accd.at[pl.ds(sid * ZDROWS, ZDROWS)])
    plsc.subcore_barrier()

    lane = lax.broadcasted_iota(jnp.int32, (L,), 0)
    perms = [jnp.bitwise_xor(lane, m) for m in (1, 2, 4, 8)]
    zero16 = jnp.zeros((L,), jnp.float32)
    one_i = jnp.ones((L,), jnp.int32)
    lane_onehot = [
        (one_i - jnp.minimum(jnp.bitwise_xor(lane, h), one_i)
         ).astype(jnp.float32)
        for h in range(H)]

    def lane_sum_splat(v):
        for p in perms:
            v = v + v.at[p].get(mode="promise_in_bounds", unique_indices=True)
        return v

    def edge_body(i, carry):
        dvec = zero16
        for h in range(H):
            sl = pl.ds(h * C, C)
            eo = k_v[i, sl] * q_v[i, sl] * ee_v[i, sl] * 0.25
            eout_v[i, sl] = eo
            exv = jnp.exp(lane_sum_splat(eo))
            rown_v[i, sl] = exv * v_v[i, sl]
            dvec = dvec + exv * lane_onehot[h]
        ib = pl.multiple_of(jnp.minimum((i // L) * L, BLK - L), 8)
        off = i - ib
        chunk = src_v[pl.ds(ib, L)]
        slotv = chunk.at[jnp.broadcast_to(off, (L,))].get(
            mode="promise_in_bounds") & 7
        for j in range(8):
            mf = (one_i - jnp.minimum(jnp.bitwise_xor(slotv, j), one_i)
                  ).astype(jnp.float32)
            rowd_v[i, pl.ds(j * L, L)] = dvec * mf
        return carry

    _windows = sorted({min(t * L, BLK - L) for t in range((BLK + L - 1) // L)})

    def conv_idx(b):
        sboff = pl.multiple_of((b % SB) * BLK, 8)
        for w in _windows:
            s = src_sb[pl.ds(sboff + w, L)]
            src_v[pl.ds(w, L)] = s
            src8_v[pl.ds(w, L)] = lax.shift_right_logical(s, 3)

    def wait_out():
        pltpu.make_async_copy(eout_v, eout_hbm.at[pl.ds(0, BLK)],
                              semo).wait()

    def blk_body(b, carry):
        base = wid * EDGES_PER_W + b * BLK

        @pl.when(b % SB == 0)
        def _():
            pltpu.sync_copy(src_hbm.at[pl.ds(base, SB * BLK)], src_sb)
            pltpu.sync_copy(dst_hbm.at[pl.ds(base, SB * BLK)], dst_sb)

        sboff = pl.multiple_of((b % SB) * BLK, 8)
        cpk = pltpu.async_copy(kh_hbm.at[src_sb.at[pl.ds(sboff, BLK)]],
                               k_v, sem0)
        cpq = pltpu.async_copy(qh_hbm.at[dst_sb.at[pl.ds(sboff, BLK)]],
                               q_v, sem1)
        cpv = pltpu.async_copy(vh_hbm.at[dst_sb.at[pl.ds(sboff, BLK)]],
                               v_v, sem2)
        pltpu.sync_copy(ee_hbm.at[pl.ds(base, BLK)], ee_v)

        @pl.when(b >= 1)
        def _():
            wait_out()

        conv_idx(b)
        cpk.wait()
        cpq.wait()
        cpv.wait()
        lax.fori_loop(0, BLK, edge_body, 0)
        pltpu.async_copy(eout_v, eout_hbm.at[pl.ds(base, BLK)], semo)
        pltpu.sync_copy(rown_v, accn.at[src_v], add=True)
        pltpu.sync_copy(rowd_v, accd.at[src8_v], add=True)
        return carry

    lax.fori_loop(0, NBLK, blk_body, 0)
    wait_out()

    plsc.subcore_barrier()
    pltpu.sync_copy(accn.at[pl.ds(sid * ZROWS, ZROWS)],
                    partn_hbm.at[cid, pl.ds(sid * ZROWS, ZROWS)])
    pltpu.sync_copy(accd.at[pl.ds(sid * ZDROWS, ZDROWS)],
                    partd_hbm.at[cid, pl.ds(sid * ZDROWS, ZDROWS)])


def _combine_body(n0, n1, d0, d1, m, out):
    num = n0[...] + n1[...]
    den = jnp.dot(d0[...] + d1[...], m[...], preferred_element_type=jnp.float32)
    out[...] = num / jnp.maximum(den, 1e-30)


def _combine(p0n, p1n, p0d, p1d, onehot):
    n = p0n.shape[0]
    bn = 2000
    grid = n // bn
    rows = pl.BlockSpec((bn, HC), lambda i: (i, 0))
    drows = pl.BlockSpec((bn, H), lambda i: (i, 0))
    return pl.pallas_call(
        _combine_body,
        grid=(grid,),
        in_specs=[rows, rows, drows, drows,
                  pl.BlockSpec((H, HC), lambda i: (0, 0))],
        out_specs=rows,
        out_shape=jax.ShapeDtypeStruct((n, HC), jnp.float32),
    )(p0n, p1n, p0d, p1d, onehot)


def kernel(x, e, edge_index, Qw, Qb, Kw, Kb, Vw, Vb, Ew, Eb):
    src = edge_index[0].astype(jnp.int32)
    dst = edge_index[1].astype(jnp.int32)

    Kh, Qh, Vh = _node_proj(x, Kw, Kb, Qw, Qb, Vw, Vb)
    Ee = _edge_proj(e, Ew, Eb)

    zrows = jnp.zeros((ZROWS, HC), jnp.float32)
    e_out, partn, partd = _edge_kernel(Kh, Qh, Vh, Ee, src, dst, zrows)

    p0n = partn[0, :N_NODES, :]
    p1n = partn[1, :N_NODES, :]
    p0d = partd[0].reshape(NDEN * 8, L)[:N_NODES, :H]
    p1d = partd[1].reshape(NDEN * 8, L)[:N_NODES, :H]
    onehot = jnp.repeat(jnp.eye(H, dtype=jnp.float32), C, axis=1)
    h_out = _combine(p0n, p1n, p0d, p1d, onehot)
    return (h_out, e_out)
